# baseline (speedup 1.0000x reference)
import functools

import jax
import jax.numpy as jnp
from jax import lax
from jax.experimental import pallas as pl
from jax.experimental.pallas import tpu as pltpu
from jax.experimental.pallas import tpu_sc as plsc

N = 10000
E = 320000
D = 128
H = 256
OUT = 128

NC = 2
NS = 16
NW = NC * NS
CHUNK = 64
TOTCH = E // CHUNK
NCHUNK = 160
W0 = 17
BLK = 8
NPAD = 10240
DUMP = NPAD - 1
RPT = NPAD // NS


def _edge_pass_body(x_hbm, src_hbm, dst_hbm, e_hbm, out_hbm,
                    src_blk, dst_blk, dst_dummy, xg_v, e_v, aggr_s,
                    gsem0, gsem1, gsem2, esem0, esem1, ssem0, ssem1, ssem2):
    cid = lax.axis_index("c")
    sid = lax.axis_index("s")
    wid = cid * NS + sid
    gsem = (gsem0, gsem1, gsem2)
    esem = (esem0, esem1)
    ssem = (ssem0, ssem1, ssem2)
    start_w = (NCHUNK * jnp.minimum(wid, W0)
               + (NCHUNK - 8) * jnp.maximum(wid - W0, 0))
    cnt_w = jnp.where(wid < W0, NCHUNK, NCHUNK - 8)

    def zrow(r, _):
        for j in range(D // 16):
            xg_v[0, r, pl.ds(j * 16, 16)] = jnp.zeros((16,), jnp.float32)
        return 0
    lax.fori_loop(0, CHUNK, zrow, 0)
    for j in range(CHUNK // 16):
        dst_dummy[pl.ds(j * 16, 16)] = jnp.full((16,), DUMP, jnp.int32)
    for k in range(RPT // CHUNK):
        pltpu.sync_copy(xg_v.at[0], aggr_s.at[pl.ds(sid * RPT + k * CHUNK, CHUNK)])
    plsc.subcore_barrier()

    def load_blk(blk, buf):
        g0 = start_w + blk * BLK
        pltpu.sync_copy(src_hbm.at[pl.ds(g0 * CHUNK, BLK * CHUNK)],
                        src_blk.at[buf])
        pltpu.sync_copy(dst_hbm.at[pl.ds(g0, BLK)],
                        dst_blk.at[pl.ds(buf * BLK, BLK)])

    def src_ref(j):
        return src_blk.at[(j // BLK) % 2, pl.ds((j % BLK) * CHUNK, CHUNK)]

    def dst_ref(j):
        return dst_blk.at[((j // BLK) % 2) * BLK + (j % BLK)]

    def e_slice(j):
        gch = jnp.minimum(start_w + j, TOTCH - 1)
        return e_hbm.at[pl.ds(gch * CHUNK, CHUNK)]

    def issue_g(j, s):
        pltpu.async_copy(x_hbm.at[src_ref(j)], xg_v.at[s], gsem[s])

    def wait_g(j, s):
        pltpu.make_async_copy(x_hbm.at[src_ref(j)], xg_v.at[s], gsem[s]).wait()

    def issue_e(j, s):
        pltpu.async_copy(e_slice(j), e_v.at[s], esem[s])

    def wait_e(j, s):
        pltpu.make_async_copy(e_slice(j), e_v.at[s], esem[s]).wait()

    def compute(s, se):
        @plsc.parallel_loop(0, CHUNK, step=1, unroll=2)
        def row(r):
            for q in range(D // 16):
                sl = pl.ds(q * 16, 16)
                xg_v[s, r, sl] = jnp.maximum(xg_v[s, r, sl] + e_v[se, r, sl], 0.0)

    def issue_s(j, s, se, may_be_dummy):
        del se
        if may_be_dummy:
            @pl.when(j < cnt_w)
            def _():
                pltpu.async_copy(xg_v.at[s], aggr_s.at[dst_ref(j)], ssem[s],
                                 add=True)

            @pl.when(j >= cnt_w)
            def _():
                pltpu.async_copy(xg_v.at[s], aggr_s.at[dst_dummy], ssem[s],
                                 add=True)
        else:
            pltpu.async_copy(xg_v.at[s], aggr_s.at[dst_ref(j)], ssem[s],
                             add=True)

    def wait_s(s):
        pltpu.make_async_copy(xg_v.at[0], aggr_s.at[dst_dummy], ssem[s]).wait()

    def step(j, gs, es, wait_prev_scatter, prefetch, maybe_blk,
             may_be_dummy=False):
        wait_g(j, gs)
        wait_e(j, es)
        compute(gs, es)
        issue_s(j, gs, es, may_be_dummy)
        if maybe_blk:
            @pl.when(lax.rem(j + 2, BLK) == 0)
            def _():
                load_blk((j + 2) // BLK, ((j + 2) // BLK) % 2)
        if prefetch:
            issue_e(j + 2, es)
        if wait_prev_scatter:
            wait_s((gs + 2) % 3)
        if prefetch:
            issue_g(j + 2, (gs + 2) % 3)

    load_blk(0, 0)
    issue_g(0, 0)
    issue_e(0, 0)
    issue_g(1, 1)
    issue_e(1, 1)
    step(0, 0, 0, False, True, False)
    step(1, 1, 1, True, True, False)
    step(2, 2, 0, True, True, False)

    def block(b, _):
        j0 = 3 + 6 * b
        for i in range(6):
            step(j0 + i, i % 3, (1 + i) % 2, True, True, True,
                 may_be_dummy=True)
        return 0
    lax.fori_loop(0, 25, block, 0)

    step(153, 0, 1, True, True, False, True)
    step(154, 1, 0, True, True, False, True)
    step(155, 2, 1, True, True, False, True)
    step(156, 0, 0, True, True, False, True)
    step(157, 1, 1, True, True, False, True)
    step(158, 2, 0, True, False, False, True)
    step(159, 0, 1, True, False, False, True)
    wait_s(0)

    plsc.subcore_barrier()
    row0 = sid * RPT
    pltpu.sync_copy(aggr_s.at[pl.ds(row0, RPT)], out_hbm.at[cid, pl.ds(row0, RPT)])


_edge_pass = pl.kernel(
    _edge_pass_body,
    out_type=jax.ShapeDtypeStruct((NC, NPAD, D), jnp.float32),
    mesh=plsc.VectorSubcoreMesh(core_axis_name="c", subcore_axis_name="s"),
    scratch_types=[
        pltpu.VMEM((2, BLK * CHUNK), jnp.int32),
        pltpu.VMEM((2 * BLK, CHUNK), jnp.int32),
        pltpu.VMEM((CHUNK,), jnp.int32),
        pltpu.VMEM((3, CHUNK, D), jnp.float32),
        pltpu.VMEM((2, CHUNK, D), jnp.float32),
        pltpu.VMEM_SHARED((NPAD, D), jnp.float32),
        pltpu.SemaphoreType.DMA,
        pltpu.SemaphoreType.DMA,
        pltpu.SemaphoreType.DMA,
        pltpu.SemaphoreType.DMA,
        pltpu.SemaphoreType.DMA,
        pltpu.SemaphoreType.DMA,
        pltpu.SemaphoreType.DMA,
        pltpu.SemaphoreType.DMA,
    ],
    name="gine_edge_pass",
)


BE = 2560


def _edge_mm_body(ea_ref, w_ref, b_ref, e_ref):
    a = ea_ref[...].astype(jnp.bfloat16)
    e_ref[...] = jnp.dot(a, w_ref[...], preferred_element_type=jnp.float32) + b_ref[...]


def _edge_mm(edge_attr, e_W, e_b):
    full = pl.BlockSpec((D, D), lambda i: (0, 0))
    vec = pl.BlockSpec((D,), lambda i: (0,))
    return pl.pallas_call(
        _edge_mm_body,
        grid=(E // BE,),
        in_specs=[pl.BlockSpec((BE, D), lambda i: (i, 0)), full, vec],
        out_specs=pl.BlockSpec((BE, D), lambda i: (i, 0)),
        out_shape=jax.ShapeDtypeStruct((E, D), jnp.float32),
    )(edge_attr, e_W, e_b)


BN = 400


def _ln(t, g, b):
    mu = jnp.mean(t, axis=-1, keepdims=True)
    var = jnp.mean((t - mu) * (t - mu), axis=-1, keepdims=True)
    return (t - mu) * lax.rsqrt(var + 1e-5) * g + b


def _node1_body(x_ref, a0_ref, a1_ref, w1_ref, b1_ref, g_ref, beta_ref,
                w2_ref, b2_ref, ng_ref, nb_ref, h1_ref):
    h = x_ref[...] + a0_ref[...] + a1_ref[...]
    t = jnp.dot(h, w1_ref[...], preferred_element_type=jnp.float32) + b1_ref[...]
    t = jnp.maximum(_ln(t, g_ref[...], beta_ref[...]), 0.0)
    u = jnp.dot(t, w2_ref[...], preferred_element_type=jnp.float32) + b2_ref[...]
    h1_ref[...] = _ln(jnp.maximum(u, 0.0), ng_ref[...], nb_ref[...])


def _node1(x, a0, a1, w1, b1, g, beta, w2, b2, ng, nb):
    blk = pl.BlockSpec((BN, D), lambda i: (i, 0))
    fw1 = pl.BlockSpec((D, H), lambda i: (0, 0))
    fw2 = pl.BlockSpec((H, D), lambda i: (0, 0))
    vH = pl.BlockSpec((H,), lambda i: (0,))
    vD = pl.BlockSpec((D,), lambda i: (0,))
    return pl.pallas_call(
        _node1_body,
        grid=(N // BN,),
        in_specs=[blk, blk, blk, fw1, vH, vH, vH, fw2, vD, vD, vD],
        out_specs=blk,
        out_shape=jax.ShapeDtypeStruct((N, D), jnp.float32),
    )(x, a0, a1, w1, b1, g, beta, w2, b2, ng, nb)


def _node2_body(h1_ref, a0_ref, a1_ref, w1_ref, b1_ref, g_ref, beta_ref,
                w2_ref, b2_ref, ng_ref, nb_ref, la_ref, lb_ref, lbias_ref,
                out_ref):
    h1 = h1_ref[...]
    h = h1 + a0_ref[...] + a1_ref[...]
    t = jnp.dot(h, w1_ref[...], preferred_element_type=jnp.float32) + b1_ref[...]
    t = jnp.maximum(_ln(t, g_ref[...], beta_ref[...]), 0.0)
    u = jnp.dot(t, w2_ref[...], preferred_element_type=jnp.float32) + b2_ref[...]
    h2 = _ln(jnp.maximum(u, 0.0), ng_ref[...], nb_ref[...])
    o = (jnp.dot(h1, la_ref[...], preferred_element_type=jnp.float32)
         + jnp.dot(h2, lb_ref[...], preferred_element_type=jnp.float32)
         + lbias_ref[...])
    out_ref[...] = jnp.maximum(o, 0.0)


def _node2(h1, a0, a1, w1, b1, g, beta, w2, b2, ng, nb, lin_Wa, lin_Wb, lin_b):
    blk = pl.BlockSpec((BN, D), lambda i: (i, 0))
    fw1 = pl.BlockSpec((D, H), lambda i: (0, 0))
    fw2 = pl.BlockSpec((H, OUT), lambda i: (0, 0))
    fl = pl.BlockSpec((D, OUT), lambda i: (0, 0))
    vH = pl.BlockSpec((H,), lambda i: (0,))
    vO = pl.BlockSpec((OUT,), lambda i: (0,))
    return pl.pallas_call(
        _node2_body,
        grid=(N // BN,),
        in_specs=[blk, blk, blk, fw1, vH, vH, vH, fw2, vO, vO, vO, fl, fl, vO],
        out_specs=pl.BlockSpec((BN, OUT), lambda i: (i, 0)),
        out_shape=jax.ShapeDtypeStruct((N, OUT), jnp.float32),
    )(h1, a0, a1, w1, b1, g, beta, w2, b2, ng, nb, lin_Wa, lin_Wb, lin_b)


def kernel(x, edge_index, edge_attr, pos,
           e1_W, e1_b, m1_W1, m1_b1, m1_g, m1_beta, m1_W2, m1_b2, n1_g, n1_b,
           e2_W, e2_b, m2_W1, m2_b1, m2_g, m2_beta, m2_W2, m2_b2, n2_g, n2_b,
           lin_W, lin_b):
    src_p = jnp.pad(edge_index[0], (0, BLK * CHUNK))
    dst_p = jnp.pad(edge_index[1].reshape(TOTCH, CHUNK), ((0, BLK), (0, 0)))

    e1 = _edge_mm(edge_attr, e1_W.astype(jnp.bfloat16), e1_b)
    aggr1 = _edge_pass(x, src_p, dst_p, e1)
    e2 = _edge_mm(edge_attr, e2_W.astype(jnp.bfloat16), e2_b)
    h1 = _node1(x, aggr1[0], aggr1[1],
                m1_W1, m1_b1, m1_g, m1_beta, m1_W2, m1_b2, n1_g, n1_b)

    aggr2 = _edge_pass(h1, src_p, dst_p, e2)
    out = _node2(h1, aggr2[0], aggr2[1],
                 m2_W1, m2_b1, m2_g, m2_beta, m2_W2, m2_b2, n2_g, n2_b,
                 lin_W[:D], lin_W[D:], lin_b)
    return out


# speedup vs baseline: 1.0080x; 1.0080x over previous
import functools

import jax
import jax.numpy as jnp
from jax import lax
from jax.experimental import pallas as pl
from jax.experimental.pallas import tpu as pltpu
from jax.experimental.pallas import tpu_sc as plsc

N = 10000
E = 320000
D = 128
H = 256
OUT = 128

NC = 2
NS = 16
NW = NC * NS
CHUNK = 64
TOTCH = E // CHUNK
NCHUNK = 160
W0 = 17
BLK = 8
NPAD = 10240
DUMP = NPAD - 1
RPT = NPAD // NS


def _edge_pass_body(x_hbm, src_hbm, dst_hbm, e_hbm, out_hbm,
                    src_blk, dst_blk, dst_dummy, xg_v, e_v, aggr_s,
                    gsem0, gsem1, gsem2, esem0, esem1, ssem0, ssem1, ssem2):
    cid = lax.axis_index("c")
    sid = lax.axis_index("s")
    wid = cid * NS + sid
    gsem = (gsem0, gsem1, gsem2)
    esem = (esem0, esem1)
    ssem = (ssem0, ssem1, ssem2)
    start_w = (NCHUNK * jnp.minimum(wid, W0)
               + (NCHUNK - 8) * jnp.maximum(wid - W0, 0))
    cnt_w = jnp.where(wid < W0, NCHUNK, NCHUNK - 8)

    def zrow(r, _):
        for j in range(D // 16):
            xg_v[0, r, pl.ds(j * 16, 16)] = jnp.zeros((16,), jnp.float32)
        return 0
    lax.fori_loop(0, CHUNK, zrow, 0)
    for j in range(CHUNK // 16):
        dst_dummy[pl.ds(j * 16, 16)] = jnp.full((16,), DUMP, jnp.int32)
    for k in range(RPT // CHUNK):
        pltpu.sync_copy(xg_v.at[0], aggr_s.at[pl.ds(sid * RPT + k * CHUNK, CHUNK)])
    plsc.subcore_barrier()

    def load_blk(blk, buf):
        g0 = start_w + blk * BLK
        pltpu.sync_copy(src_hbm.at[pl.ds(g0 * CHUNK, BLK * CHUNK)],
                        src_blk.at[buf])
        pltpu.sync_copy(dst_hbm.at[pl.ds(g0, BLK)],
                        dst_blk.at[pl.ds(buf * BLK, BLK)])

    def src_ref(j):
        return src_blk.at[(j // BLK) % 2, pl.ds((j % BLK) * CHUNK, CHUNK)]

    def dst_ref(j):
        return dst_blk.at[((j // BLK) % 2) * BLK + (j % BLK)]

    def e_slice(j):
        gch = jnp.minimum(start_w + j, TOTCH - 1)
        return e_hbm.at[pl.ds(gch * CHUNK, CHUNK)]

    def issue_g(j, s):
        pltpu.async_copy(x_hbm.at[src_ref(j)], xg_v.at[s], gsem[s])

    def wait_g(j, s):
        pltpu.make_async_copy(x_hbm.at[src_ref(j)], xg_v.at[s], gsem[s]).wait()

    def issue_e(j, s):
        pltpu.async_copy(e_slice(j), e_v.at[s], esem[s])

    def wait_e(j, s):
        pltpu.make_async_copy(e_slice(j), e_v.at[s], esem[s]).wait()

    def compute(s, se):
        def row(r, _):
            for q in range(D // 16):
                sl = pl.ds(q * 16, 16)
                xg_v[s, r, sl] = jnp.maximum(xg_v[s, r, sl] + e_v[se, r, sl], 0.0)
            return 0
        lax.fori_loop(0, CHUNK, row, 0)

    def issue_s(j, s, se, may_be_dummy):
        del se
        if may_be_dummy:
            @pl.when(j < cnt_w)
            def _():
                pltpu.async_copy(xg_v.at[s], aggr_s.at[dst_ref(j)], ssem[s],
                                 add=True)

            @pl.when(j >= cnt_w)
            def _():
                pltpu.async_copy(xg_v.at[s], aggr_s.at[dst_dummy], ssem[s],
                                 add=True)
        else:
            pltpu.async_copy(xg_v.at[s], aggr_s.at[dst_ref(j)], ssem[s],
                             add=True)

    def wait_s(s):
        pltpu.make_async_copy(xg_v.at[0], aggr_s.at[dst_dummy], ssem[s]).wait()

    def step(j, gs, es, wait_prev_scatter, prefetch, maybe_blk,
             may_be_dummy=False):
        wait_g(j, gs)
        wait_e(j, es)
        compute(gs, es)
        issue_s(j, gs, es, may_be_dummy)
        if maybe_blk:
            @pl.when(lax.rem(j + 2, BLK) == 0)
            def _():
                load_blk((j + 2) // BLK, ((j + 2) // BLK) % 2)
        if prefetch:
            issue_e(j + 2, es)
        if wait_prev_scatter:
            wait_s((gs + 2) % 3)
        if prefetch:
            issue_g(j + 2, (gs + 2) % 3)

    load_blk(0, 0)
    issue_g(0, 0)
    issue_e(0, 0)
    issue_g(1, 1)
    issue_e(1, 1)
    step(0, 0, 0, False, True, False)
    step(1, 1, 1, True, True, False)
    step(2, 2, 0, True, True, False)

    def block(b, _):
        j0 = 3 + 6 * b
        for i in range(6):
            step(j0 + i, i % 3, (1 + i) % 2, True, True, True,
                 may_be_dummy=True)
        return 0
    lax.fori_loop(0, 25, block, 0)

    step(153, 0, 1, True, True, False, True)
    step(154, 1, 0, True, True, False, True)
    step(155, 2, 1, True, True, False, True)
    step(156, 0, 0, True, True, False, True)
    step(157, 1, 1, True, True, False, True)
    step(158, 2, 0, True, False, False, True)
    step(159, 0, 1, True, False, False, True)
    wait_s(0)

    plsc.subcore_barrier()
    row0 = sid * RPT
    pltpu.sync_copy(aggr_s.at[pl.ds(row0, RPT)], out_hbm.at[cid, pl.ds(row0, RPT)])


_edge_pass = pl.kernel(
    _edge_pass_body,
    out_type=jax.ShapeDtypeStruct((NC, NPAD, D), jnp.float32),
    mesh=plsc.VectorSubcoreMesh(core_axis_name="c", subcore_axis_name="s"),
    scratch_types=[
        pltpu.VMEM((2, BLK * CHUNK), jnp.int32),
        pltpu.VMEM((2 * BLK, CHUNK), jnp.int32),
        pltpu.VMEM((CHUNK,), jnp.int32),
        pltpu.VMEM((3, CHUNK, D), jnp.float32),
        pltpu.VMEM((2, CHUNK, D), jnp.float32),
        pltpu.VMEM_SHARED((NPAD, D), jnp.float32),
        pltpu.SemaphoreType.DMA,
        pltpu.SemaphoreType.DMA,
        pltpu.SemaphoreType.DMA,
        pltpu.SemaphoreType.DMA,
        pltpu.SemaphoreType.DMA,
        pltpu.SemaphoreType.DMA,
        pltpu.SemaphoreType.DMA,
        pltpu.SemaphoreType.DMA,
    ],
    name="gine_edge_pass",
)


BE = 2560


def _edge_mm_body(ea_ref, w_ref, b_ref, e_ref):
    a = ea_ref[...].astype(jnp.bfloat16)
    e_ref[...] = jnp.dot(a, w_ref[...], preferred_element_type=jnp.float32) + b_ref[...]


def _edge_mm(edge_attr, e_W, e_b):
    full = pl.BlockSpec((D, D), lambda i: (0, 0))
    vec = pl.BlockSpec((D,), lambda i: (0,))
    return pl.pallas_call(
        _edge_mm_body,
        grid=(E // BE,),
        in_specs=[pl.BlockSpec((BE, D), lambda i: (i, 0)), full, vec],
        out_specs=pl.BlockSpec((BE, D), lambda i: (i, 0)),
        out_shape=jax.ShapeDtypeStruct((E, D), jnp.float32),
    )(edge_attr, e_W, e_b)


BN = 400


def _ln(t, g, b):
    mu = jnp.mean(t, axis=-1, keepdims=True)
    var = jnp.mean((t - mu) * (t - mu), axis=-1, keepdims=True)
    return (t - mu) * lax.rsqrt(var + 1e-5) * g + b


def _node1_body(x_ref, a0_ref, a1_ref, w1_ref, b1_ref, g_ref, beta_ref,
                w2_ref, b2_ref, ng_ref, nb_ref, h1_ref):
    h = x_ref[...] + a0_ref[...] + a1_ref[...]
    t = jnp.dot(h, w1_ref[...], preferred_element_type=jnp.float32) + b1_ref[...]
    t = jnp.maximum(_ln(t, g_ref[...], beta_ref[...]), 0.0)
    u = jnp.dot(t, w2_ref[...], preferred_element_type=jnp.float32) + b2_ref[...]
    h1_ref[...] = _ln(jnp.maximum(u, 0.0), ng_ref[...], nb_ref[...])


def _node1(x, a0, a1, w1, b1, g, beta, w2, b2, ng, nb):
    blk = pl.BlockSpec((BN, D), lambda i: (i, 0))
    fw1 = pl.BlockSpec((D, H), lambda i: (0, 0))
    fw2 = pl.BlockSpec((H, D), lambda i: (0, 0))
    vH = pl.BlockSpec((H,), lambda i: (0,))
    vD = pl.BlockSpec((D,), lambda i: (0,))
    return pl.pallas_call(
        _node1_body,
        grid=(N // BN,),
        in_specs=[blk, blk, blk, fw1, vH, vH, vH, fw2, vD, vD, vD],
        out_specs=blk,
        out_shape=jax.ShapeDtypeStruct((N, D), jnp.float32),
    )(x, a0, a1, w1, b1, g, beta, w2, b2, ng, nb)


def _node2_body(h1_ref, a0_ref, a1_ref, w1_ref, b1_ref, g_ref, beta_ref,
                w2_ref, b2_ref, ng_ref, nb_ref, la_ref, lb_ref, lbias_ref,
                out_ref):
    h1 = h1_ref[...]
    h = h1 + a0_ref[...] + a1_ref[...]
    t = jnp.dot(h, w1_ref[...], preferred_element_type=jnp.float32) + b1_ref[...]
    t = jnp.maximum(_ln(t, g_ref[...], beta_ref[...]), 0.0)
    u = jnp.dot(t, w2_ref[...], preferred_element_type=jnp.float32) + b2_ref[...]
    h2 = _ln(jnp.maximum(u, 0.0), ng_ref[...], nb_ref[...])
    o = (jnp.dot(h1, la_ref[...], preferred_element_type=jnp.float32)
         + jnp.dot(h2, lb_ref[...], preferred_element_type=jnp.float32)
         + lbias_ref[...])
    out_ref[...] = jnp.maximum(o, 0.0)


def _node2(h1, a0, a1, w1, b1, g, beta, w2, b2, ng, nb, lin_Wa, lin_Wb, lin_b):
    blk = pl.BlockSpec((BN, D), lambda i: (i, 0))
    fw1 = pl.BlockSpec((D, H), lambda i: (0, 0))
    fw2 = pl.BlockSpec((H, OUT), lambda i: (0, 0))
    fl = pl.BlockSpec((D, OUT), lambda i: (0, 0))
    vH = pl.BlockSpec((H,), lambda i: (0,))
    vO = pl.BlockSpec((OUT,), lambda i: (0,))
    return pl.pallas_call(
        _node2_body,
        grid=(N // BN,),
        in_specs=[blk, blk, blk, fw1, vH, vH, vH, fw2, vO, vO, vO, fl, fl, vO],
        out_specs=pl.BlockSpec((BN, OUT), lambda i: (i, 0)),
        out_shape=jax.ShapeDtypeStruct((N, OUT), jnp.float32),
    )(h1, a0, a1, w1, b1, g, beta, w2, b2, ng, nb, lin_Wa, lin_Wb, lin_b)


def kernel(x, edge_index, edge_attr, pos,
           e1_W, e1_b, m1_W1, m1_b1, m1_g, m1_beta, m1_W2, m1_b2, n1_g, n1_b,
           e2_W, e2_b, m2_W1, m2_b1, m2_g, m2_beta, m2_W2, m2_b2, n2_g, n2_b,
           lin_W, lin_b):
    src_p = jnp.pad(edge_index[0], (0, BLK * CHUNK))
    dst_p = jnp.pad(edge_index[1].reshape(TOTCH, CHUNK), ((0, BLK), (0, 0)))

    e1 = _edge_mm(edge_attr, e1_W.astype(jnp.bfloat16), e1_b)
    aggr1 = _edge_pass(x, src_p, dst_p, e1)
    e2 = _edge_mm(edge_attr, e2_W.astype(jnp.bfloat16), e2_b)
    h1 = _node1(x, aggr1[0], aggr1[1],
                m1_W1, m1_b1, m1_g, m1_beta, m1_W2, m1_b2, n1_g, n1_b)

    aggr2 = _edge_pass(h1, src_p, dst_p, e2)
    out = _node2(h1, aggr2[0], aggr2[1],
                 m2_W1, m2_b1, m2_g, m2_beta, m2_W2, m2_b2, n2_g, n2_b,
                 lin_W[:D], lin_W[D:], lin_b)
    return out
